# baseline (speedup 1.0000x reference)
import functools

import jax
import jax.numpy as jnp
from jax import lax
from jax.experimental import pallas as pl
from jax.experimental.pallas import tpu as pltpu
from jax.experimental.pallas import tpu_sc as plsc

N_USERS = 100000
N_ITEMS = 100000
K = 64
B = 4096
L = 16
NC, NS = 2, 16
NW = NC * NS
BPW = B // NW
G = BPW // L


def _sc_body(u_hbm, i_hbm, bi_hbm, bu_hbm, qa_hbm, qb_hbm, pa_hbm, pb_hbm,
             uvec_hbm,
             out_hbm,
             u_v, i_v, qidx, pidx, qi_vals, pu_vals, bu_v, bi_v, u_const,
             out_v, sem_b, sem_q, sem_p):
    wid = lax.axis_index("s") * NC + lax.axis_index("c")
    base = wid * BPW

    pltpu.sync_copy(u_hbm.at[pl.ds(base, BPW)], u_v)
    pltpu.sync_copy(i_hbm.at[pl.ds(base, BPW)], i_v)
    pltpu.sync_copy(uvec_hbm, u_const)

    cp_bu = pltpu.async_copy(bu_hbm.at[u_v], bu_v, sem_b)
    cp_bi = pltpu.async_copy(bi_hbm.at[i_v], bi_v, sem_b)

    KH = K // 2

    def build_and_fire(kk, _):
        off = kk * N_ITEMS
        for g in range(G):
            sl = pl.ds(g * L, L)
            qidx[kk, sl] = i_v[sl] + off
            pidx[kk + KH, sl] = u_v[sl] + off
        pltpu.async_copy(qa_hbm.at[qidx.at[kk]], qi_vals.at[kk], sem_q)
        pltpu.async_copy(pb_hbm.at[pidx.at[kk + KH]],
                         pu_vals.at[kk + KH], sem_p)
        return 0

    def build_and_fire2(kk, _):
        off = kk * N_ITEMS
        for g in range(G):
            sl = pl.ds(g * L, L)
            qidx[kk + KH, sl] = i_v[sl] + off
            pidx[kk, sl] = u_v[sl] + off
        pltpu.async_copy(qb_hbm.at[qidx.at[kk + KH]],
                         qi_vals.at[kk + KH], sem_q)
        pltpu.async_copy(pa_hbm.at[pidx.at[kk]], pu_vals.at[kk], sem_p)
        return 0

    lax.fori_loop(0, KH, build_and_fire, 0, unroll=False)
    lax.fori_loop(0, KH, build_and_fire2, 0, unroll=False)

    cp_bu.wait()
    cp_bi.wait()

    def drain(kk, _):
        pltpu.make_async_copy(
            qa_hbm.at[qidx.at[kk]], qi_vals.at[kk], sem_q).wait()
        pltpu.make_async_copy(
            pa_hbm.at[pidx.at[kk]], pu_vals.at[kk], sem_p).wait()
        return 0

    lax.fori_loop(0, 2 * KH, drain, 0, unroll=False)

    def dot_step(kk, accs):
        out = []
        for g in range(G):
            sl = pl.ds(g * L, L)
            out.append(accs[g] + pu_vals[kk, sl] * qi_vals[kk, sl])
        return tuple(out)

    accs = lax.fori_loop(
        0, K, dot_step,
        tuple(jnp.zeros((L,), jnp.float32) for _ in range(G)),
        unroll=False)

    uc = u_const[...]
    for g in range(G):
        sl = pl.ds(g * L, L)
        out_v[sl] = uc + bu_v[sl] + bi_v[sl] + accs[g]
    pltpu.sync_copy(out_v, out_hbm.at[pl.ds(base, BPW)])


@jax.jit
def _run(u, i, bi, bu, qa, qb, pa, pb, u_vec):
    mesh = plsc.VectorSubcoreMesh(core_axis_name="c", subcore_axis_name="s")
    kfn = functools.partial(
        pl.kernel,
        mesh=mesh,
        compiler_params=pltpu.CompilerParams(
            needs_layout_passes=False, use_tc_tiling_on_sc=False),
        out_type=jax.ShapeDtypeStruct((B,), jnp.float32),
        scratch_types=[
            pltpu.VMEM((BPW,), jnp.int32),
            pltpu.VMEM((BPW,), jnp.int32),
            pltpu.VMEM((K, BPW), jnp.int32),
            pltpu.VMEM((K, BPW), jnp.int32),
            pltpu.VMEM((K, BPW), jnp.float32),
            pltpu.VMEM((K, BPW), jnp.float32),
            pltpu.VMEM((BPW,), jnp.float32),
            pltpu.VMEM((BPW,), jnp.float32),
            pltpu.VMEM((L,), jnp.float32),
            pltpu.VMEM((BPW,), jnp.float32),
            pltpu.SemaphoreType.DMA,
            pltpu.SemaphoreType.DMA,
            pltpu.SemaphoreType.DMA,
        ],
    )(_sc_body)
    return kfn(u, i, bi, bu, qa, qb, pa, pb, u_vec)


def kernel(u, i, bi, bu, qi, pu, U):
    puT = pu.T
    qa = qi[:K // 2].reshape(-1)
    qb = qi[K // 2:].reshape(-1)
    pa = puT[:K // 2].reshape(-1)
    pb = puT[K // 2:].reshape(-1)
    u_vec = jnp.full((L,), U, jnp.float32)
    return _run(u, i, bi, bu, qa, qb, pa, pb, u_vec)


# speedup vs baseline: 1.4013x; 1.4013x over previous
import functools

import jax
import jax.numpy as jnp
from jax import lax
from jax.experimental import pallas as pl
from jax.experimental.pallas import tpu as pltpu
from jax.experimental.pallas import tpu_sc as plsc

N_USERS = 100000
N_ITEMS = 100000
K = 64
B = 4096
L = 16
NC, NS = 2, 16
NW = NC * NS
BPW = B // NW
G = BPW // L

_params = pltpu.CompilerParams(
    needs_layout_passes=False, use_tc_tiling_on_sc=False)
_mesh = plsc.VectorSubcoreMesh(core_axis_name="c", subcore_axis_name="s")


def _sc_body_a(u_hbm, puflat_hbm, puv_hbm,
               u_v, pidx, pu_vals, sem_p):
    wid = lax.axis_index("s") * NC + lax.axis_index("c")
    base = wid * BPW

    pltpu.sync_copy(u_hbm.at[pl.ds(base, BPW)], u_v)

    def build_and_fire(kk, _):
        off = kk * N_USERS
        for g in range(G):
            sl = pl.ds(g * L, L)
            pidx[kk, sl] = u_v[sl] + off
        pltpu.async_copy(puflat_hbm.at[pidx.at[kk]], pu_vals.at[kk], sem_p)
        return 0

    lax.fori_loop(0, K, build_and_fire, 0, unroll=False)

    def drain(kk, _):
        pltpu.make_async_copy(
            puflat_hbm.at[pidx.at[kk]], pu_vals.at[kk], sem_p).wait()
        return 0

    lax.fori_loop(0, K, drain, 0, unroll=False)

    pltpu.sync_copy(pu_vals, puv_hbm.at[wid])


def _sc_body_b(i_hbm, u_hbm, bi_hbm, bu_hbm, qiflat_hbm, puv_hbm, uvec_hbm,
               out_hbm,
               i_v, u_v, qidx, qi_vals, pu_vals, bu_v, bi_v, u_const,
               out_v, sem_b, sem_q, sem_s):
    wid = lax.axis_index("s") * NC + lax.axis_index("c")
    base = wid * BPW

    pltpu.sync_copy(i_hbm.at[pl.ds(base, BPW)], i_v)
    pltpu.sync_copy(u_hbm.at[pl.ds(base, BPW)], u_v)
    pltpu.sync_copy(uvec_hbm, u_const)

    cp_bu = pltpu.async_copy(bu_hbm.at[u_v], bu_v, sem_b)
    cp_bi = pltpu.async_copy(bi_hbm.at[i_v], bi_v, sem_b)
    cp_pu = pltpu.async_copy(puv_hbm.at[wid], pu_vals, sem_s)

    def build_and_fire(kk, _):
        off = kk * N_ITEMS
        for g in range(G):
            sl = pl.ds(g * L, L)
            qidx[kk, sl] = i_v[sl] + off
        pltpu.async_copy(qiflat_hbm.at[qidx.at[kk]], qi_vals.at[kk], sem_q)
        return 0

    lax.fori_loop(0, K, build_and_fire, 0, unroll=False)

    cp_bu.wait()
    cp_bi.wait()
    cp_pu.wait()

    def drain(kk, _):
        pltpu.make_async_copy(
            qiflat_hbm.at[qidx.at[kk]], qi_vals.at[kk], sem_q).wait()
        return 0

    lax.fori_loop(0, K, drain, 0, unroll=False)

    def dot_step(kk, accs):
        out = []
        for g in range(G):
            sl = pl.ds(g * L, L)
            out.append(accs[g] + pu_vals[kk, sl] * qi_vals[kk, sl])
        return tuple(out)

    accs = lax.fori_loop(
        0, K, dot_step,
        tuple(jnp.zeros((L,), jnp.float32) for _ in range(G)),
        unroll=False)

    uc = u_const[...]
    for g in range(G):
        sl = pl.ds(g * L, L)
        out_v[sl] = uc + bu_v[sl] + bi_v[sl] + accs[g]
    pltpu.sync_copy(out_v, out_hbm.at[pl.ds(base, BPW)])


@jax.jit
def _run(u, i, bi, bu, qi_flat, pu_flat, u_vec):
    ka = functools.partial(
        pl.kernel,
        mesh=_mesh,
        compiler_params=_params,
        out_type=jax.ShapeDtypeStruct((NW, K, BPW), jnp.float32),
        scratch_types=[
            pltpu.VMEM((BPW,), jnp.int32),
            pltpu.VMEM((K, BPW), jnp.int32),
            pltpu.VMEM((K, BPW), jnp.float32),
            pltpu.SemaphoreType.DMA,
        ],
    )(_sc_body_a)
    puv = ka(u, pu_flat)

    kb = functools.partial(
        pl.kernel,
        mesh=_mesh,
        compiler_params=_params,
        out_type=jax.ShapeDtypeStruct((B,), jnp.float32),
        scratch_types=[
            pltpu.VMEM((BPW,), jnp.int32),
            pltpu.VMEM((BPW,), jnp.int32),
            pltpu.VMEM((K, BPW), jnp.int32),
            pltpu.VMEM((K, BPW), jnp.float32),
            pltpu.VMEM((K, BPW), jnp.float32),
            pltpu.VMEM((BPW,), jnp.float32),
            pltpu.VMEM((BPW,), jnp.float32),
            pltpu.VMEM((L,), jnp.float32),
            pltpu.VMEM((BPW,), jnp.float32),
            pltpu.SemaphoreType.DMA,
            pltpu.SemaphoreType.DMA,
            pltpu.SemaphoreType.DMA,
        ],
    )(_sc_body_b)
    return kb(i, u, bi, bu, qi_flat, puv, u_vec)


def kernel(u, i, bi, bu, qi, pu, U):
    qi_flat = qi.reshape(-1)
    pu_flat = pu.T.reshape(-1)
    u_vec = jnp.full((L,), U, jnp.float32)
    return _run(u, i, bi, bu, qi_flat, pu_flat, u_vec)
